# baseline (speedup 1.0000x reference)
import functools

import jax
import jax.numpy as jnp
from jax import lax
from jax.experimental import pallas as pl
from jax.experimental.pallas import tpu as pltpu
from jax.experimental.pallas import tpu_sc as plsc

BATCH = 4096
NUM_SLOTS = 520
VOCAB = 1000000
EMB = 16

NUM_WORKERS = 32
ROWS_PER_WORKER = BATCH // NUM_WORKERS
CHUNK_ROWS = 2
CHUNK_SLOTS = CHUNK_ROWS * NUM_SLOTS
NUM_CHUNKS = ROWS_PER_WORKER // CHUNK_ROWS
SUPERCHUNKS = ROWS_PER_WORKER // 16
W_ROWS = VOCAB // EMB


W_PACK_WORKERS = 25
W_PACK_ROWS = W_ROWS // W_PACK_WORKERS


def _w16_body(w_hbm, w16_hbm, wflat_v, wtile_v, sem):
    num_cores = 2
    wid = lax.axis_index("s") * num_cores + lax.axis_index("c")

    @pl.when(wid < W_PACK_WORKERS)
    def _():
        base = wid * W_PACK_ROWS * EMB
        pltpu.sync_copy(w_hbm.at[pl.ds(base, W_PACK_ROWS * EMB)], wflat_v)

        def row_body(r, _):
            wtile_v[r, :] = wflat_v[pl.ds(r * 16, 16)]
            return 0
        lax.fori_loop(0, W_PACK_ROWS, row_body, 0)
        pltpu.sync_copy(wtile_v, w16_hbm.at[pl.ds(wid * W_PACK_ROWS,
                                                  W_PACK_ROWS)])


@jax.jit
def _w16_restage(w_flat):
    mesh = plsc.VectorSubcoreMesh(core_axis_name="c", subcore_axis_name="s")
    return pl.kernel(
        _w16_body,
        out_type=jax.ShapeDtypeStruct((W_ROWS, EMB), jnp.float32),
        mesh=mesh,
        compiler_params=pltpu.CompilerParams(use_tc_tiling_on_sc=False,
                                             needs_layout_passes=False),
        scratch_types=[
            pltpu.VMEM((W_PACK_ROWS * EMB,), jnp.float32),
            pltpu.VMEM((W_PACK_ROWS, EMB), jnp.float32),
            pltpu.SemaphoreType.DMA,
        ],
    )(w_flat)


def _fm_body(vals_hbm, w16_hbm, vtab_hbm, idx_hbm, out_hbm,
             idx_v0, idx_v1, idxhi_v0, idxhi_v1, val_v0, val_v1,
             vrows_v0, vrows_v1, wrows_v0, wrows_v1, out_v,
             sem_v0, sem_v1, sem_w0, sem_w1):
    num_cores = 2
    wid = lax.axis_index("s") * num_cores + lax.axis_index("c")
    lane_iota = lax.iota(jnp.int32, 16)

    bufs = [
        (idx_v0, idxhi_v0, val_v0, vrows_v0, wrows_v0, sem_v0, sem_w0),
        (idx_v1, idxhi_v1, val_v1, vrows_v1, wrows_v1, sem_v1, sem_w1),
    ]

    def fire(gc, b):
        idx_b, idxhi_b, val_b, vr_b, wr_b, sv, sw = bufs[b]
        base = wid * ROWS_PER_WORKER * NUM_SLOTS + gc * CHUNK_SLOTS
        pltpu.sync_copy(idx_hbm.at[pl.ds(base, CHUNK_SLOTS)], idx_b)
        pltpu.sync_copy(vals_hbm.at[pl.ds(base, CHUNK_SLOTS)], val_b)
        pltpu.async_copy(vtab_hbm.at[idx_b], vr_b, sv)

        def hi_body(g, _):
            s = g * 16
            idxhi_b[pl.ds(s, 16)] = idx_b[pl.ds(s, 16)] >> 4
            return 0
        lax.fori_loop(0, CHUNK_SLOTS // 16, hi_body, 0)
        pltpu.async_copy(w16_hbm.at[idxhi_b], wr_b, sw)

    def drain(b):
        _, _, _, vr_b, wr_b, sv, sw = bufs[b]
        pltpu.make_async_copy(
            vtab_hbm.at[pl.ds(0, CHUNK_SLOTS)], vr_b, sv).wait()
        pltpu.make_async_copy(
            w16_hbm.at[pl.ds(0, CHUNK_SLOTS)], wr_b, sw).wait()

    def lane_sum(x):
        for sh in (8, 4, 2, 1):
            perm = lane_iota ^ sh
            x = x + x.at[perm].get(mode="promise_in_bounds")
        return x

    z = jnp.zeros((16,), jnp.float32)

    def slot_update(row, wrow, lok, valk, acc, sq, ex):
        t = row * valk
        acc = acc + t
        sq = sq + t * t
        wsel = jnp.where(lane_iota == lok, wrow, 0.0)
        ex = ex + wsel * valk
        return acc, sq, ex

    fire(0, 0)

    def superchunk_body(sc, _):
        outvec = jnp.zeros((16,), jnp.float32)
        for sub in range(8):
            p = sub % 2
            _, _, val_b, vr_b, wr_b, _, _ = bufs[p]
            gc = sc * 8 + sub
            idx_b = bufs[p][0]
            drain(p)
            if sub < 7:
                fire(gc + 1, 1 - p)
            else:
                @pl.when(sc < SUPERCHUNKS - 1)
                def _():
                    fire(gc + 1, 1 - p)

            def half_row(base, carry0):
                acc0, sq0, ex0 = carry0

                def group(g, carry):
                    a0, a1, q0, q1, e0, e1 = carry
                    s0 = base + g * 16
                    valvec = val_b[pl.ds(s0, 16)]
                    lovec = idx_b[pl.ds(s0, 16)] & 15
                    accs = [a0, a1]
                    sqs = [q0, q1]
                    exs = [e0, e1]
                    for k in range(16):
                        j = k % 2
                        accs[j], sqs[j], exs[j] = slot_update(
                            vr_b[s0 + k, :], wr_b[s0 + k, :],
                            lovec[k], valvec[k],
                            accs[j], sqs[j], exs[j])
                    return (*accs, *sqs, *exs)

                a0, a1, q0, q1, e0, e1 = lax.fori_loop(
                    0, 32, group, (acc0, z, sq0, z, ex0, z))
                return a0 + a1, q0 + q1, e0 + e1

            accA, sqA, exA = half_row(0, (z, z, z))
            accB, sqB, exB = z, z, z
            sm = 512
            valvec = val_b[pl.ds(sm, 16)]
            lovec = idx_b[pl.ds(sm, 16)] & 15
            for k in range(16):
                if k < 8:
                    accA, sqA, exA = slot_update(
                        vr_b[sm + k, :], wr_b[sm + k, :],
                        lovec[k], valvec[k], accA, sqA, exA)
                else:
                    accB, sqB, exB = slot_update(
                        vr_b[sm + k, :], wr_b[sm + k, :],
                        lovec[k], valvec[k], accB, sqB, exB)
            accB, sqB, exB = half_row(528, (accB, sqB, exB))

            for (acc, sq, ex, lane) in (
                    (accA, sqA, exA, sub * 2),
                    (accB, sqB, exB, sub * 2 + 1)):
                combined = 0.5 * (acc * acc - sq) + ex
                total = lane_sum(combined)
                outvec = jnp.where(lane_iota == lane, total, outvec)
        out_v[pl.ds(sc * 16, 16)] = outvec
        return 0

    lax.fori_loop(0, SUPERCHUNKS, superchunk_body, 0)
    pltpu.sync_copy(out_v, out_hbm.at[pl.ds(wid * ROWS_PER_WORKER,
                                            ROWS_PER_WORKER)])


@jax.jit
def _fm_sc(vals_flat, w_first, v_second, idx_flat):
    w16 = _w16_restage(w_first.reshape(-1))
    mesh = plsc.VectorSubcoreMesh(core_axis_name="c", subcore_axis_name="s")
    return pl.kernel(
        _fm_body,
        out_type=jax.ShapeDtypeStruct((BATCH,), jnp.float32),
        mesh=mesh,
        compiler_params=pltpu.CompilerParams(use_tc_tiling_on_sc=False,
                                             needs_layout_passes=False),
        scratch_types=[
            pltpu.VMEM((CHUNK_SLOTS,), jnp.int32),
            pltpu.VMEM((CHUNK_SLOTS,), jnp.int32),
            pltpu.VMEM((CHUNK_SLOTS,), jnp.int32),
            pltpu.VMEM((CHUNK_SLOTS,), jnp.int32),
            pltpu.VMEM((CHUNK_SLOTS,), jnp.float32),
            pltpu.VMEM((CHUNK_SLOTS,), jnp.float32),
            pltpu.VMEM((CHUNK_SLOTS, EMB), jnp.float32),
            pltpu.VMEM((CHUNK_SLOTS, EMB), jnp.float32),
            pltpu.VMEM((CHUNK_SLOTS, EMB), jnp.float32),
            pltpu.VMEM((CHUNK_SLOTS, EMB), jnp.float32),
            pltpu.VMEM((ROWS_PER_WORKER,), jnp.float32),
            pltpu.SemaphoreType.DMA,
            pltpu.SemaphoreType.DMA,
            pltpu.SemaphoreType.DMA,
            pltpu.SemaphoreType.DMA,
        ],
    )(vals_flat, w16, v_second, idx_flat)


def kernel(feature_values, w_first, v_second, fm_bias, feature_idx):
    idx_flat = feature_idx.astype(jnp.int32).reshape(-1)
    vals_flat = feature_values.reshape(-1)
    logits = _fm_sc(vals_flat, w_first, v_second, idx_flat)
    return logits + fm_bias[0]


# speedup vs baseline: 1.0721x; 1.0721x over previous
import functools

import jax
import jax.numpy as jnp
from jax import lax
from jax.experimental import pallas as pl
from jax.experimental.pallas import tpu as pltpu
from jax.experimental.pallas import tpu_sc as plsc

BATCH = 4096
NUM_SLOTS = 520
VOCAB = 1000000
EMB = 16

NUM_WORKERS = 32
ROWS_PER_WORKER = BATCH // NUM_WORKERS
CHUNK_ROWS = 4
CHUNK_SLOTS = CHUNK_ROWS * NUM_SLOTS
NUM_CHUNKS = ROWS_PER_WORKER // CHUNK_ROWS
PAIR_SLOTS = 2 * NUM_SLOTS
SUPERCHUNKS = ROWS_PER_WORKER // 16


def _fm_body(vals_hbm, w_hbm, vtab_hbm, idx_hbm, out_hbm,
             idx_v0, idx_v1, val_v0, val_v1, w_v0, w_v1,
             vrows_v0, vrows_v1, out_v,
             sem_v0, sem_v1, sem_w0, sem_w1):
    num_cores = 2
    wid = lax.axis_index("s") * num_cores + lax.axis_index("c")
    lane_iota = lax.iota(jnp.int32, 16)

    bufs = [
        (idx_v0, val_v0, w_v0, vrows_v0, sem_v0, sem_w0),
        (idx_v1, val_v1, w_v1, vrows_v1, sem_v1, sem_w1),
    ]

    def fire(gc, b):
        idx_b, val_b, w_b, vr_b, sv, sw = bufs[b]
        base = wid * ROWS_PER_WORKER * NUM_SLOTS + gc * CHUNK_SLOTS
        pltpu.sync_copy(idx_hbm.at[pl.ds(base, CHUNK_SLOTS)], idx_b)
        pltpu.sync_copy(vals_hbm.at[pl.ds(base, CHUNK_SLOTS)], val_b)
        pltpu.async_copy(vtab_hbm.at[idx_b], vr_b, sv)
        pltpu.async_copy(w_hbm.at[idx_b], w_b, sw)

    def drain(b):
        _, _, w_b, vr_b, sv, sw = bufs[b]
        pltpu.make_async_copy(
            vtab_hbm.at[pl.ds(0, CHUNK_SLOTS)], vr_b, sv).wait()
        pltpu.make_async_copy(
            w_hbm.at[pl.ds(0, CHUNK_SLOTS)], w_b, sw).wait()

    def lane_sum(x):
        for sh in (8, 4, 2, 1):
            perm = lane_iota ^ sh
            x = x + x.at[perm].get(mode="promise_in_bounds")
        return x

    z = jnp.zeros((16,), jnp.float32)

    fire(0, 0)

    def superchunk_body(sc, _):
        outvec = jnp.zeros((16,), jnp.float32)
        for sub in range(4):
            p = sub % 2
            _, val_b, w_b, vr_b, _, _ = bufs[p]
            gc = sc * 4 + sub
            drain(p)
            if sub < 3:
                fire(gc + 1, 1 - p)
            else:
                @pl.when(sc < SUPERCHUNKS - 1)
                def _():
                    fire(gc + 1, 1 - p)

            def half_row(base, carry0):
                acc0, sq0, fv0 = carry0

                def group(g, carry):
                    a0, a1, a2, a3, q0, q1, q2, q3, fv = carry
                    s0 = base + g * 16
                    valvec = val_b[pl.ds(s0, 16)]
                    wvec = w_b[pl.ds(s0, 16)]
                    fv = fv + wvec * valvec
                    accs = [a0, a1, a2, a3]
                    sqs = [q0, q1, q2, q3]
                    for k in range(16):
                        row = vr_b[s0 + k, :]
                        t = row * valvec[k]
                        accs[k % 4] = accs[k % 4] + t
                        sqs[k % 4] = sqs[k % 4] + t * t
                    return (*accs, *sqs, fv)

                a0, a1, a2, a3, q0, q1, q2, q3, fv = lax.fori_loop(
                    0, 32, group, (acc0, z, z, z, sq0, z, z, z, fv0))
                return (a0 + a1) + (a2 + a3), (q0 + q1) + (q2 + q3), fv

            for pair in range(2):
                pbase = pair * PAIR_SLOTS
                accA, sqA, fvA = half_row(pbase, (z, z, z))
                accB, sqB, fvB = z, z, z
                sm = pbase + 512
                valvec = val_b[pl.ds(sm, 16)]
                wvec = w_b[pl.ds(sm, 16)]
                wv = wvec * valvec
                fvA = fvA + jnp.where(lane_iota < 8, wv, 0.0)
                fvB = fvB + jnp.where(lane_iota < 8, 0.0, wv)
                for k in range(16):
                    row = vr_b[sm + k, :]
                    t = row * valvec[k]
                    if k < 8:
                        accA = accA + t
                        sqA = sqA + t * t
                    else:
                        accB = accB + t
                        sqB = sqB + t * t
                accB, sqB, fvB = half_row(pbase + 528, (accB, sqB, fvB))

                for (acc, sq, fv, lane) in (
                        (accA, sqA, fvA, sub * 4 + pair * 2),
                        (accB, sqB, fvB, sub * 4 + pair * 2 + 1)):
                    combined = 0.5 * (acc * acc - sq) + fv
                    total = lane_sum(combined)
                    outvec = jnp.where(lane_iota == lane, total, outvec)
        out_v[pl.ds(sc * 16, 16)] = outvec
        return 0

    lax.fori_loop(0, SUPERCHUNKS, superchunk_body, 0)
    pltpu.sync_copy(out_v, out_hbm.at[pl.ds(wid * ROWS_PER_WORKER,
                                            ROWS_PER_WORKER)])


@jax.jit
def _fm_sc(vals_flat, w_flat, v_second, idx_flat):
    mesh = plsc.VectorSubcoreMesh(core_axis_name="c", subcore_axis_name="s")
    return pl.kernel(
        _fm_body,
        out_type=jax.ShapeDtypeStruct((BATCH,), jnp.float32),
        mesh=mesh,
        compiler_params=pltpu.CompilerParams(use_tc_tiling_on_sc=False),
        scratch_types=[
            pltpu.VMEM((CHUNK_SLOTS,), jnp.int32),
            pltpu.VMEM((CHUNK_SLOTS,), jnp.int32),
            pltpu.VMEM((CHUNK_SLOTS,), jnp.float32),
            pltpu.VMEM((CHUNK_SLOTS,), jnp.float32),
            pltpu.VMEM((CHUNK_SLOTS,), jnp.float32),
            pltpu.VMEM((CHUNK_SLOTS,), jnp.float32),
            pltpu.VMEM((CHUNK_SLOTS, EMB), jnp.float32),
            pltpu.VMEM((CHUNK_SLOTS, EMB), jnp.float32),
            pltpu.VMEM((ROWS_PER_WORKER,), jnp.float32),
            pltpu.SemaphoreType.DMA,
            pltpu.SemaphoreType.DMA,
            pltpu.SemaphoreType.DMA,
            pltpu.SemaphoreType.DMA,
        ],
    )(vals_flat, w_flat, v_second, idx_flat)


def kernel(feature_values, w_first, v_second, fm_bias, feature_idx):
    idx_flat = feature_idx.astype(jnp.int32).reshape(-1)
    vals_flat = feature_values.reshape(-1)
    w_flat = w_first.reshape(-1)
    logits = _fm_sc(vals_flat, w_flat, v_second, idx_flat)
    return logits + fm_bias[0]
